# baseline (speedup 1.0000x reference)
import functools

import jax
import jax.numpy as jnp
from jax import lax
from jax.experimental import pallas as pl
from jax.experimental.pallas import tpu as pltpu
from jax.experimental.pallas import tpu_sc as plsc

ROWS = 16384
COLS = 20
VOCAB = 14
EMB = 32
B = ROWS * COLS

NC, NS = 2, 16
NW = NC * NS

W = 128
K = 10
C = W * K
G = B // (NW * C)
NBUF = 2

_MESH = plsc.VectorSubcoreMesh(core_axis_name="c", subcore_axis_name="s")


@jax.jit
def _sc_gather(table, idx2d):
    @functools.partial(
        pl.kernel,
        out_type=jax.ShapeDtypeStruct((B, EMB), jnp.float32),
        mesh=_MESH,
        compiler_params=pltpu.CompilerParams(use_tc_tiling_on_sc=False),
        scratch_types=[
            pltpu.VMEM((NBUF, K, W), jnp.int32),
            pltpu.VMEM((NBUF, C, EMB), jnp.float32),
            pltpu.VMEM_SHARED((VOCAB, EMB), jnp.float32),
            pltpu.SemaphoreType.DMA,
            pltpu.SemaphoreType.DMA,
            pltpu.SemaphoreType.DMA,
            pltpu.SemaphoreType.DMA,
            pltpu.SemaphoreType.DMA,
        ],
    )
    def k(table_hbm, idx_hbm, out_hbm, idx_v, rows_v, table_v,
          g_sem, o_sem0, o_sem1, i_sem0, i_sem1):
        wid = lax.axis_index("s") * NC + lax.axis_index("c")
        chunk0 = wid * G
        o_sems = (o_sem0, o_sem1)
        i_sems = (i_sem0, i_sem1)

        @pl.when(lax.axis_index("s") == 0)
        def _():
            pltpu.sync_copy(table_hbm, table_v)

        plsc.subcore_barrier()

        for b in range(NBUF):
            pltpu.async_copy(
                idx_hbm.at[pl.ds((chunk0 + b) * K, K)], idx_v.at[b],
                i_sems[b],
            )

        @pl.loop(0, G, step=NBUF)
        def _(g):
            for b in range(NBUF):
                chunk = chunk0 + g + b
                @pl.when(g + b >= NBUF)
                def _():
                    pltpu.make_async_copy(
                        rows_v.at[b], out_hbm.at[pl.ds(chunk * C, C)],
                        o_sems[b],
                    ).wait()

                pltpu.make_async_copy(
                    idx_hbm.at[pl.ds(chunk * K, K)], idx_v.at[b], i_sems[b]
                ).wait()

                handles = [
                    pltpu.async_copy(
                        table_v.at[idx_v.at[b, j]],
                        rows_v.at[b, pl.ds(j * W, W)],
                        g_sem,
                    )
                    for j in range(K)
                ]
                for h in handles:
                    h.wait()

                @pl.when(g + b + NBUF < G)
                def _():
                    pltpu.async_copy(
                        idx_hbm.at[pl.ds((chunk + NBUF) * K, K)],
                        idx_v.at[b], i_sems[b],
                    )

                pltpu.async_copy(
                    rows_v.at[b], out_hbm.at[pl.ds(chunk * C, C)],
                    o_sems[b],
                )

        for b in range(NBUF):
            chunk = chunk0 + G - NBUF + b
            pltpu.make_async_copy(
                rows_v.at[b], out_hbm.at[pl.ds(chunk * C, C)], o_sems[b]
            ).wait()

    return k(table, idx2d)


def kernel(x, synth_emb_weight):
    perm = (
        jnp.arange(160, dtype=jnp.int32)
        .reshape(8, 5, 4)
        .transpose(1, 0, 2)
        .reshape(160)
    )
    idx = (
        jnp.take(x.astype(jnp.int8).reshape(ROWS // 8, 160), perm, axis=1)
        .reshape(B)
        .astype(jnp.int32)
        .reshape(B // W, W)
    )
    out = _sc_gather(synth_emb_weight, idx)
    return (
        out.reshape(ROWS // 8, 5, 8, 4 * EMB)
        .transpose(0, 2, 1, 3)
        .reshape(ROWS, COLS * EMB)
    )


# speedup vs baseline: 1.0065x; 1.0065x over previous
import functools

import jax
import jax.numpy as jnp
from jax import lax
from jax.experimental import pallas as pl
from jax.experimental.pallas import tpu as pltpu
from jax.experimental.pallas import tpu_sc as plsc

ROWS = 16384
COLS = 20
VOCAB = 14
EMB = 32
B = ROWS * COLS

NC, NS = 2, 16
NW = NC * NS

W = 128
K = 5
C = W * K
G = B // (NW * C)
NBUF = 4

_MESH = plsc.VectorSubcoreMesh(core_axis_name="c", subcore_axis_name="s")


@jax.jit
def _sc_gather(table, idx2d):
    @functools.partial(
        pl.kernel,
        out_type=jax.ShapeDtypeStruct((B, EMB), jnp.float32),
        mesh=_MESH,
        compiler_params=pltpu.CompilerParams(use_tc_tiling_on_sc=False),
        scratch_types=[
            pltpu.VMEM((NBUF, K, W), jnp.int32),
            pltpu.VMEM((NBUF, C, EMB), jnp.float32),
            pltpu.VMEM_SHARED((VOCAB, EMB), jnp.float32),
            pltpu.SemaphoreType.DMA,
            pltpu.SemaphoreType.DMA,
            pltpu.SemaphoreType.DMA,
            pltpu.SemaphoreType.DMA,
            pltpu.SemaphoreType.DMA,
            pltpu.SemaphoreType.DMA,
            pltpu.SemaphoreType.DMA,
            pltpu.SemaphoreType.DMA,
            pltpu.SemaphoreType.DMA,
        ],
    )
    def k(table_hbm, idx_hbm, out_hbm, idx_v, rows_v, table_v,
          g_sem, o_sem0, o_sem1, o_sem2, o_sem3,
          i_sem0, i_sem1, i_sem2, i_sem3):
        wid = lax.axis_index("s") * NC + lax.axis_index("c")
        chunk0 = wid * G
        o_sems = (o_sem0, o_sem1, o_sem2, o_sem3)
        i_sems = (i_sem0, i_sem1, i_sem2, i_sem3)

        @pl.when(lax.axis_index("s") == 0)
        def _():
            pltpu.sync_copy(table_hbm, table_v)

        plsc.subcore_barrier()

        for b in range(NBUF):
            pltpu.async_copy(
                idx_hbm.at[pl.ds((chunk0 + b) * K, K)], idx_v.at[b],
                i_sems[b],
            )

        @pl.loop(0, G, step=NBUF)
        def _(g):
            for b in range(NBUF):
                chunk = chunk0 + g + b
                @pl.when(g + b >= NBUF)
                def _():
                    pltpu.make_async_copy(
                        rows_v.at[b], out_hbm.at[pl.ds(chunk * C, C)],
                        o_sems[b],
                    ).wait()

                pltpu.make_async_copy(
                    idx_hbm.at[pl.ds(chunk * K, K)], idx_v.at[b], i_sems[b]
                ).wait()

                handles = [
                    pltpu.async_copy(
                        table_v.at[idx_v.at[b, j]],
                        rows_v.at[b, pl.ds(j * W, W)],
                        g_sem,
                    )
                    for j in range(K)
                ]
                for h in handles:
                    h.wait()

                @pl.when(g + b + NBUF < G)
                def _():
                    pltpu.async_copy(
                        idx_hbm.at[pl.ds((chunk + NBUF) * K, K)],
                        idx_v.at[b], i_sems[b],
                    )

                pltpu.async_copy(
                    rows_v.at[b], out_hbm.at[pl.ds(chunk * C, C)],
                    o_sems[b],
                )

        for b in range(NBUF):
            chunk = chunk0 + G - NBUF + b
            pltpu.make_async_copy(
                rows_v.at[b], out_hbm.at[pl.ds(chunk * C, C)], o_sems[b]
            ).wait()

    return k(table, idx2d)


def kernel(x, synth_emb_weight):
    perm = (
        jnp.arange(160, dtype=jnp.int32)
        .reshape(8, 5, 4)
        .transpose(1, 0, 2)
        .reshape(160)
    )
    idx = (
        jnp.take(x.astype(jnp.int8).reshape(ROWS // 8, 160), perm, axis=1)
        .reshape(B)
        .astype(jnp.int32)
        .reshape(B // W, W)
    )
    out = _sc_gather(synth_emb_weight, idx)
    return (
        out.reshape(ROWS // 8, 5, 8, 4 * EMB)
        .transpose(0, 2, 1, 3)
        .reshape(ROWS, COLS * EMB)
    )
